# Initial kernel scaffold; baseline (speedup 1.0000x reference)
import jax
import jax.numpy as jnp
from jax.experimental import pallas as pl


def kernel(input, t_p):
    raise NotImplementedError("write your pallas kernel here")


# speedup vs baseline: 7.7032x; 7.7032x over previous
import jax
import jax.numpy as jnp
from jax import lax
from jax.experimental import pallas as pl


def _body(x_ref, tp_ref, o_ref):
    p, c = x_ref.shape[1], x_ref.shape[2]
    x = x_ref[0]
    tp = tp_ref[0]
    m = jnp.max(x, axis=0, keepdims=True)
    q_iota = lax.broadcasted_iota(jnp.int32, (p, c), 0)
    cand = jnp.where(x == m, q_iota, p)
    idx = jnp.min(cand, axis=0, keepdims=True)
    onehot = (q_iota == idx).astype(jnp.float32)
    tmpl_t = lax.dot_general(tp, onehot, (((0,), (0,)), ((), ())),
                             preferred_element_type=jnp.float32)
    o_ref[0] = jnp.maximum(x * tmpl_t, 0.0)


def kernel(input, t_p):
    b, h, w, c = input.shape
    p = h * w
    x = input.reshape(b, p, c)
    tp = t_p.reshape(b, p, p)
    out = pl.pallas_call(
        _body,
        grid=(b,),
        in_specs=[pl.BlockSpec((1, p, c), lambda i: (i, 0, 0)),
                  pl.BlockSpec((1, p, p), lambda i: (i, 0, 0))],
        out_specs=pl.BlockSpec((1, p, c), lambda i: (i, 0, 0)),
        out_shape=jax.ShapeDtypeStruct((b, p, c), jnp.float32),
    )(x, tp)
    return out.reshape(b, h, w, c)
